# baseline (speedup 1.0000x reference)
import functools

import jax
import jax.numpy as jnp
from jax import lax
from jax.experimental import pallas as pl
from jax.experimental.pallas import tpu as pltpu
from jax.experimental.pallas import tpu_sc as plsc

N = 10000
E = 320000
D = 128
NC = 2
NS = 16
NW = NC * NS
EPT = E // NW
C = 80
NCHUNK = EPT // C
ZCH = 80
NZ = N // ZCH
VPR = D // 16


def _edge_body(h_hbm, src_hbm, dst_hbm, attr_hbm, out_hbm,
               sidx, di0, di1, gb0, gb1, abuf, agg_sh,
               gs0, gs1, asem, ds0, ds1):
    c = lax.axis_index("c")
    s = lax.axis_index("s")
    wid = s * NC + c
    base = wid * EPT
    nz_mine = (NZ - s + NS - 1) // NS

    gbufs = (gb0, gb1)
    didxs = (di0, di1)
    gsems = (gs0, gs1)
    dsems = (ds0, ds1)

    pltpu.sync_copy(src_hbm.at[pl.ds(base, EPT)], sidx)

    def _zrow(i, carry):
        for j in range(VPR):
            gb0[i, pl.ds(j * 16, 16)] = jnp.zeros((16,), jnp.float32)
        return carry
    lax.fori_loop(0, ZCH, _zrow, 0)

    def _zcopy(k, carry):
        r = (s + k * NS) * ZCH
        pltpu.sync_copy(gb0, agg_sh.at[pl.ds(r, ZCH)])
        return carry
    lax.fori_loop(0, nz_mine, _zcopy, 0)
    plsc.subcore_barrier()

    def _issue_gd(q, b):
        pltpu.async_copy(dst_hbm.at[pl.ds(base + q * C, C)], didxs[b], dsems[b])
        pltpu.async_copy(h_hbm.at[sidx.at[pl.ds(q * C, C)]], gbufs[b], gsems[b])

    def _issue_attr(q):
        pltpu.async_copy(attr_hbm.at[pl.ds((base + q * C) * (D // 2), C * D // 2)],
                         abuf, asem)

    def _step(q, b, prefetch):
        if prefetch:
            _issue_gd(q + 1, b ^ 1)
        pltpu.make_async_copy(
            attr_hbm.at[pl.ds((base + q * C) * (D // 2), C * D // 2)],
            abuf, asem).wait()
        pltpu.make_async_copy(h_hbm.at[sidx.at[pl.ds(q * C, C)]],
                              gbufs[b], gsems[b]).wait()
        gbuf = gbufs[b]

        def _row(i, rc):
            for j in range(VPR // 2):
                u = abuf[pl.ds(i * (D // 2) + 16 * j, 16)]
                ea = lax.bitcast_convert_type(u << 16, jnp.float32)
                eb = lax.bitcast_convert_type(u & jnp.int32(-65536), jnp.float32)
                c0 = 32 * j
                gbuf[i, pl.ds(c0, 16)] = jnp.maximum(
                    gbuf[i, pl.ds(c0, 16)] + ea, 0.0)
                gbuf[i, pl.ds(c0 + 16, 16)] = jnp.maximum(
                    gbuf[i, pl.ds(c0 + 16, 16)] + eb, 0.0)
            return rc
        lax.fori_loop(0, C, _row, 0)
        if prefetch:
            _issue_attr(q + 1)
        pltpu.make_async_copy(dst_hbm.at[pl.ds(base + q * C, C)],
                              didxs[b], dsems[b]).wait()
        pltpu.sync_copy(gbuf, agg_sh.at[didxs[b]], add=True)

    _issue_gd(0, 0)
    _issue_attr(0)

    def _super(j, carry):
        _step(2 * j, 0, True)
        _step(2 * j + 1, 1, True)
        return carry
    lax.fori_loop(0, (NCHUNK - 1) // 2, _super, 0)
    _step(NCHUNK - 1, 0, False)
    plsc.subcore_barrier()

    def _ocopy(k, carry):
        r = (s + k * NS) * ZCH
        pltpu.sync_copy(agg_sh.at[pl.ds(r, ZCH)],
                        out_hbm.at[pl.ds(c * N + r, ZCH)])
        return carry
    lax.fori_loop(0, nz_mine, _ocopy, 0)


@functools.lru_cache(maxsize=None)
def _get_edge_agg():
    return pl.kernel(
        _edge_body,
        out_type=jax.ShapeDtypeStruct((2 * N, D), jnp.float32),
        mesh=plsc.VectorSubcoreMesh(core_axis_name="c", subcore_axis_name="s"),
        scratch_types=[
            pltpu.VMEM((EPT,), jnp.int32),
            pltpu.VMEM((C,), jnp.int32),
            pltpu.VMEM((C,), jnp.int32),
            pltpu.VMEM((C, D), jnp.float32),
            pltpu.VMEM((C, D), jnp.float32),
            pltpu.VMEM((C * D // 2,), jnp.int32),
            pltpu.VMEM_SHARED((N, D), jnp.float32),
            pltpu.SemaphoreType.DMA,
            pltpu.SemaphoreType.DMA,
            pltpu.SemaphoreType.DMA,
            pltpu.SemaphoreType.DMA,
            pltpu.SemaphoreType.DMA,
        ],
    )


def _dense_body(x_ref, agg_ref, eps_ref, w1_ref, b1_ref, g_ref, be_ref,
                w2_ref, b2_ref, o_ref):
    h = (1.0 + eps_ref[0, 0]) * x_ref[...] + agg_ref[0:N] + agg_ref[N:2 * N]
    h1 = jnp.dot(h, w1_ref[...], preferred_element_type=jnp.float32) + b1_ref[...]
    mu = jnp.mean(h1, axis=0, keepdims=True)
    var = jnp.mean(jnp.square(h1 - mu), axis=0, keepdims=True)
    hn = (h1 - mu) * (g_ref[...] * lax.rsqrt(var + 1e-5)) + be_ref[...]
    hr = jnp.maximum(hn, 0.0)
    h2 = jnp.dot(hr, w2_ref[...], preferred_element_type=jnp.float32) + b2_ref[...]
    o_ref[...] = jnp.maximum(h2, 0.0)


_dense = pl.pallas_call(
    _dense_body,
    out_shape=jax.ShapeDtypeStruct((N, D), jnp.float32),
)


def _final_body(h0_ref, h1_ref, h2_ref, h3_ref, wl_ref, bl_ref, o_ref):
    acc = jnp.dot(h0_ref[...], wl_ref[0:D], preferred_element_type=jnp.float32)
    acc += jnp.dot(h1_ref[...], wl_ref[D:2 * D], preferred_element_type=jnp.float32)
    acc += jnp.dot(h2_ref[...], wl_ref[2 * D:3 * D], preferred_element_type=jnp.float32)
    acc += jnp.dot(h3_ref[...], wl_ref[3 * D:4 * D], preferred_element_type=jnp.float32)
    o_ref[...] = acc + bl_ref[...]


_final = pl.pallas_call(
    _final_body,
    out_shape=jax.ShapeDtypeStruct((N, D), jnp.float32),
)


def kernel(x, edge_index, edge_attr,
           W1_0, b1_0, gamma_0, beta_0, W2_0, b2_0, eps_0,
           W1_1, b1_1, gamma_1, beta_1, W2_1, b2_1, eps_1,
           W1_2, b1_2, gamma_2, beta_2, W2_2, b2_2, eps_2,
           W_lin, b_lin):
    src = edge_index[0]
    dst = edge_index[1]
    attr_bf = (edge_attr.reshape(E, D // 32, 2, 16)
               .transpose(0, 1, 3, 2).astype(jnp.bfloat16))
    attr_pk = lax.bitcast_convert_type(attr_bf, jnp.int32).reshape(E * D // 2)
    params = [
        (W1_0, b1_0, gamma_0, beta_0, W2_0, b2_0, eps_0),
        (W1_1, b1_1, gamma_1, beta_1, W2_1, b2_1, eps_1),
        (W1_2, b1_2, gamma_2, beta_2, W2_2, b2_2, eps_2),
    ]
    h = x
    hs = [x]
    for l in range(3):
        W1, b1, gamma, beta, W2, b2, eps = params[l]
        agg = _get_edge_agg()(h, src, dst, attr_pk)
        h = _dense(h, agg, eps.reshape(1, 1), W1, b1.reshape(1, 2 * D),
                   gamma.reshape(1, 2 * D), beta.reshape(1, 2 * D),
                   W2, b2.reshape(1, D))
        hs.append(h)
    return _final(hs[0], hs[1], hs[2], hs[3], W_lin, b_lin.reshape(1, D))


# speedup vs baseline: 3.5703x; 3.5703x over previous
import functools

import jax
import jax.numpy as jnp
from jax import lax
from jax.experimental import pallas as pl
from jax.experimental.pallas import tpu as pltpu
from jax.experimental.pallas import tpu_sc as plsc

N = 10000
E = 320000
D = 128
NC = 2
NS = 16
NW = NC * NS
EPT = E // NW
C = 40
NCHUNK = EPT // C
R = 4
ZCH = 40
NZ = N // ZCH
VPR = D // 16


def _edge_body(h_hbm, src_hbm, dst_hbm, attr_hbm, out_hbm, *rest):
    sib = rest[0:R]
    dib = rest[R:2 * R]
    gbufs = rest[2 * R:3 * R]
    abufs = rest[3 * R:4 * R]
    agg_sh = rest[4 * R]
    isems = rest[4 * R + 1:5 * R + 1]
    dsems = rest[5 * R + 1:6 * R + 1]
    gsems = rest[6 * R + 1:7 * R + 1]
    asems = rest[7 * R + 1:8 * R + 1]
    ssems = rest[8 * R + 1:9 * R + 1]

    c = lax.axis_index("c")
    s = lax.axis_index("s")
    wid = s * NC + c
    base = wid * EPT
    nz_mine = (NZ - s + NS - 1) // NS

    zb = gbufs[0]

    def _zrow(i, carry):
        for j in range(VPR):
            zb[i, pl.ds(j * 16, 16)] = jnp.zeros((16,), jnp.float32)
        return carry
    lax.fori_loop(0, ZCH, _zrow, 0)

    def _zcopy(k, carry):
        r = (s + k * NS) * ZCH
        pltpu.sync_copy(zb, agg_sh.at[pl.ds(r, ZCH)])
        return carry
    lax.fori_loop(0, nz_mine, _zcopy, 0)
    plsc.subcore_barrier()

    def _i_sidx(q, r):
        pltpu.async_copy(src_hbm.at[pl.ds(base + q * C, C)], sib[r], isems[r])

    def _w_sidx(q, r):
        pltpu.make_async_copy(src_hbm.at[pl.ds(base + q * C, C)],
                              sib[r], isems[r]).wait()

    def _i_didx(q, r):
        pltpu.async_copy(dst_hbm.at[pl.ds(base + q * C, C)], dib[r], dsems[r])

    def _w_didx(q, r):
        pltpu.make_async_copy(dst_hbm.at[pl.ds(base + q * C, C)],
                              dib[r], dsems[r]).wait()

    def _i_attr(q, r):
        pltpu.async_copy(attr_hbm.at[pl.ds(base + q * C, C)], abufs[r], asems[r])

    def _w_attr(q, r):
        pltpu.make_async_copy(attr_hbm.at[pl.ds(base + q * C, C)],
                              abufs[r], asems[r]).wait()

    def _i_gather(r):
        pltpu.async_copy(h_hbm.at[sib[r]], gbufs[r], gsems[r])

    def _w_gather(r):
        pltpu.make_async_copy(h_hbm.at[sib[r]], gbufs[r], gsems[r]).wait()

    def _i_scatter(r):
        pltpu.async_copy(gbufs[r], agg_sh.at[dib[r]], ssems[r], add=True)

    def _w_scatter(r):
        pltpu.make_async_copy(gbufs[r], agg_sh.at[dib[r]], ssems[r]).wait()

    def _step(q, r, pre3, pre2, w_sc):
        r3 = (r + 3) % R
        r2 = (r + 2) % R
        if pre3:
            _i_sidx(q + 3, r3)
        if w_sc:
            _w_scatter(r2)
        if pre2:
            _i_didx(q + 2, r2)
            _w_sidx(q + 2, r2)
            _i_attr(q + 2, r2)
            _i_gather(r2)
        _w_gather(r)
        _w_attr(q, r)
        gbuf, abuf = gbufs[r], abufs[r]

        def _row(i, rc):
            for j in range(VPR):
                v = gbuf[i, pl.ds(j * 16, 16)] + abuf[i, pl.ds(j * 16, 16)]
                gbuf[i, pl.ds(j * 16, 16)] = jnp.maximum(v, 0.0)
            return rc
        lax.fori_loop(0, C, _row, 0)
        _w_didx(q, r)
        _i_scatter(r)

    _i_sidx(0, 0)
    _i_sidx(1, 1)
    _i_sidx(2, 2)
    _i_didx(0, 0)
    _i_didx(1, 1)
    _i_attr(0, 0)
    _i_attr(1, 1)
    _w_sidx(0, 0)
    _i_gather(0)
    _w_sidx(1, 1)
    _i_gather(1)
    _step(0, 0, True, True, False)
    _step(1, 1, True, True, False)
    _step(2, 2, True, True, True)
    _step(3, 3, True, True, True)

    def _super(j, carry):
        q0 = R * j
        for i in range(R):
            _step(q0 + i, i, True, True, True)
        return carry
    lax.fori_loop(1, NCHUNK // R - 1, _super, 0)
    for q in range((NCHUNK // R - 1) * R, NCHUNK):
        _step(q, q % R, q + 3 < NCHUNK, q + 2 < NCHUNK, True)
    _w_scatter((NCHUNK - 2) % R)
    _w_scatter((NCHUNK - 1) % R)
    plsc.subcore_barrier()

    def _ocopy(k, carry):
        r = (s + k * NS) * ZCH
        pltpu.sync_copy(agg_sh.at[pl.ds(r, ZCH)],
                        out_hbm.at[pl.ds(c * N + r, ZCH)])
        return carry
    lax.fori_loop(0, nz_mine, _ocopy, 0)


@functools.lru_cache(maxsize=None)
def _get_edge_agg():
    return pl.kernel(
        _edge_body,
        out_type=jax.ShapeDtypeStruct((2 * N, D), jnp.float32),
        mesh=plsc.VectorSubcoreMesh(core_axis_name="c", subcore_axis_name="s"),
        scratch_types=(
            [pltpu.VMEM((C,), jnp.int32) for _ in range(2 * R)]
            + [pltpu.VMEM((C, D), jnp.float32) for _ in range(2 * R)]
            + [pltpu.VMEM_SHARED((N, D), jnp.float32)]
            + [pltpu.SemaphoreType.DMA for _ in range(5 * R)]
        ),
    )


def _dense_body(x_ref, agg_ref, eps_ref, w1_ref, b1_ref, g_ref, be_ref,
                w2_ref, b2_ref, o_ref):
    h = (1.0 + eps_ref[0, 0]) * x_ref[...] + agg_ref[0:N] + agg_ref[N:2 * N]
    h1 = jnp.dot(h, w1_ref[...], preferred_element_type=jnp.float32) + b1_ref[...]
    mu = jnp.mean(h1, axis=0, keepdims=True)
    var = jnp.mean(jnp.square(h1 - mu), axis=0, keepdims=True)
    hn = (h1 - mu) * (g_ref[...] * lax.rsqrt(var + 1e-5)) + be_ref[...]
    hr = jnp.maximum(hn, 0.0)
    h2 = jnp.dot(hr, w2_ref[...], preferred_element_type=jnp.float32) + b2_ref[...]
    o_ref[...] = jnp.maximum(h2, 0.0)


_dense = pl.pallas_call(
    _dense_body,
    out_shape=jax.ShapeDtypeStruct((N, D), jnp.float32),
)


def _final_body(h0_ref, h1_ref, h2_ref, h3_ref, wl_ref, bl_ref, o_ref):
    acc = jnp.dot(h0_ref[...], wl_ref[0:D], preferred_element_type=jnp.float32)
    acc += jnp.dot(h1_ref[...], wl_ref[D:2 * D], preferred_element_type=jnp.float32)
    acc += jnp.dot(h2_ref[...], wl_ref[2 * D:3 * D], preferred_element_type=jnp.float32)
    acc += jnp.dot(h3_ref[...], wl_ref[3 * D:4 * D], preferred_element_type=jnp.float32)
    o_ref[...] = acc + bl_ref[...]


_final = pl.pallas_call(
    _final_body,
    out_shape=jax.ShapeDtypeStruct((N, D), jnp.float32),
)


def kernel(x, edge_index, edge_attr,
           W1_0, b1_0, gamma_0, beta_0, W2_0, b2_0, eps_0,
           W1_1, b1_1, gamma_1, beta_1, W2_1, b2_1, eps_1,
           W1_2, b1_2, gamma_2, beta_2, W2_2, b2_2, eps_2,
           W_lin, b_lin):
    src = edge_index[0]
    dst = edge_index[1]
    params = [
        (W1_0, b1_0, gamma_0, beta_0, W2_0, b2_0, eps_0),
        (W1_1, b1_1, gamma_1, beta_1, W2_1, b2_1, eps_1),
        (W1_2, b1_2, gamma_2, beta_2, W2_2, b2_2, eps_2),
    ]
    h = x
    hs = [x]
    for l in range(3):
        W1, b1, gamma, beta, W2, b2, eps = params[l]
        agg = _get_edge_agg()(h, src, dst, edge_attr)
        h = _dense(h, agg, eps.reshape(1, 1), W1, b1.reshape(1, 2 * D),
                   gamma.reshape(1, 2 * D), beta.reshape(1, 2 * D),
                   W2, b2.reshape(1, D))
        hs.append(h)
    return _final(hs[0], hs[1], hs[2], hs[3], W_lin, b_lin.reshape(1, D))


# speedup vs baseline: 3.7169x; 1.0411x over previous
import functools

import jax
import jax.numpy as jnp
from jax import lax
from jax.experimental import pallas as pl
from jax.experimental.pallas import tpu as pltpu
from jax.experimental.pallas import tpu_sc as plsc

N = 10000
E = 320000
D = 128
NC = 2
NS = 16
NW = NC * NS
EPT = E // NW
C = 40
NCHUNK = EPT // C
R = 4
ZCH = 40
NZ = N // ZCH
VPR = D // 16


def _edge_body(h_hbm, src_hbm, dst_hbm, attr_hbm, out_hbm, *rest):
    sib = rest[0:R]
    dib = rest[R:2 * R]
    gbufs = rest[2 * R:3 * R]
    abufs = rest[3 * R:4 * R]
    agg_sh = rest[4 * R]
    isems = rest[4 * R + 1:5 * R + 1]
    dsems = rest[5 * R + 1:6 * R + 1]
    gsems = rest[6 * R + 1:7 * R + 1]
    asems = rest[7 * R + 1:8 * R + 1]
    ssems = rest[8 * R + 1:9 * R + 1]

    c = lax.axis_index("c")
    s = lax.axis_index("s")
    wid = s * NC + c
    base = wid * EPT
    nz_mine = (NZ - s + NS - 1) // NS

    zb = gbufs[0]
    zsem = isems[0]

    def _zrow(i, carry):
        for j in range(VPR):
            zb[i, pl.ds(j * 16, 16)] = jnp.zeros((16,), jnp.float32)
        return carry
    lax.fori_loop(0, ZCH, _zrow, 0)

    def _zcopy(k, carry):
        r = (s + k * NS) * ZCH
        pltpu.async_copy(zb, agg_sh.at[pl.ds(r, ZCH)], zsem)
        return carry
    lax.fori_loop(0, nz_mine, _zcopy, 0)

    def _zdrain(k, carry):
        r = (s + k * NS) * ZCH
        pltpu.make_async_copy(zb, agg_sh.at[pl.ds(r, ZCH)], zsem).wait()
        return carry
    lax.fori_loop(0, nz_mine, _zdrain, 0)
    plsc.subcore_barrier()

    def _i_sidx(q, r):
        pltpu.async_copy(src_hbm.at[pl.ds(base + q * C, C)], sib[r], isems[r])

    def _w_sidx(q, r):
        pltpu.make_async_copy(src_hbm.at[pl.ds(base + q * C, C)],
                              sib[r], isems[r]).wait()

    def _i_didx(q, r):
        pltpu.async_copy(dst_hbm.at[pl.ds(base + q * C, C)], dib[r], dsems[r])

    def _w_didx(q, r):
        pltpu.make_async_copy(dst_hbm.at[pl.ds(base + q * C, C)],
                              dib[r], dsems[r]).wait()

    def _i_attr(q, r):
        pltpu.async_copy(attr_hbm.at[pl.ds(base + q * C, C)], abufs[r], asems[r])

    def _w_attr(q, r):
        pltpu.make_async_copy(attr_hbm.at[pl.ds(base + q * C, C)],
                              abufs[r], asems[r]).wait()

    def _i_gather(r):
        pltpu.async_copy(h_hbm.at[sib[r]], gbufs[r], gsems[r])

    def _w_gather(r):
        pltpu.make_async_copy(h_hbm.at[sib[r]], gbufs[r], gsems[r]).wait()

    def _i_scatter(r):
        pltpu.async_copy(gbufs[r], agg_sh.at[dib[r]], ssems[r], add=True)

    def _w_scatter(r):
        pltpu.make_async_copy(gbufs[r], agg_sh.at[dib[r]], ssems[r]).wait()

    def _step(q, r, pre3, pre2, w_sc):
        r3 = (r + 3) % R
        r2 = (r + 2) % R
        if pre3:
            _i_sidx(q + 3, r3)
        if w_sc:
            _w_scatter(r2)
        if pre2:
            _i_didx(q + 2, r2)
            _w_sidx(q + 2, r2)
            _i_attr(q + 2, r2)
            _i_gather(r2)
        _w_gather(r)
        _w_attr(q, r)
        gbuf, abuf = gbufs[r], abufs[r]

        def _row(i, rc):
            for j in range(VPR):
                v = gbuf[i, pl.ds(j * 16, 16)] + abuf[i, pl.ds(j * 16, 16)]
                gbuf[i, pl.ds(j * 16, 16)] = jnp.maximum(v, 0.0)
            return rc
        lax.fori_loop(0, C, _row, 0)
        _w_didx(q, r)
        _i_scatter(r)

    _i_sidx(0, 0)
    _i_sidx(1, 1)
    _i_sidx(2, 2)
    _i_didx(0, 0)
    _i_didx(1, 1)
    _i_attr(0, 0)
    _i_attr(1, 1)
    _w_sidx(0, 0)
    _i_gather(0)
    _w_sidx(1, 1)
    _i_gather(1)
    _step(0, 0, True, True, False)
    _step(1, 1, True, True, False)
    _step(2, 2, True, True, True)
    _step(3, 3, True, True, True)

    def _super(j, carry):
        q0 = R * j
        for i in range(R):
            _step(q0 + i, i, True, True, True)
        return carry
    lax.fori_loop(1, NCHUNK // R - 1, _super, 0)
    for q in range((NCHUNK // R - 1) * R, NCHUNK):
        _step(q, q % R, q + 3 < NCHUNK, q + 2 < NCHUNK, True)
    _w_scatter((NCHUNK - 2) % R)
    _w_scatter((NCHUNK - 1) % R)
    plsc.subcore_barrier()

    osem = isems[1]

    def _ocopy(k, carry):
        r = (s + k * NS) * ZCH
        pltpu.async_copy(agg_sh.at[pl.ds(r, ZCH)],
                         out_hbm.at[pl.ds(c * N + r, ZCH)], osem)
        return carry
    lax.fori_loop(0, nz_mine, _ocopy, 0)

    def _odrain(k, carry):
        r = (s + k * NS) * ZCH
        pltpu.make_async_copy(agg_sh.at[pl.ds(r, ZCH)],
                              out_hbm.at[pl.ds(c * N + r, ZCH)], osem).wait()
        return carry
    lax.fori_loop(0, nz_mine, _odrain, 0)


@functools.lru_cache(maxsize=None)
def _get_edge_agg():
    return pl.kernel(
        _edge_body,
        out_type=jax.ShapeDtypeStruct((2 * N, D), jnp.float32),
        mesh=plsc.VectorSubcoreMesh(core_axis_name="c", subcore_axis_name="s"),
        scratch_types=(
            [pltpu.VMEM((C,), jnp.int32) for _ in range(2 * R)]
            + [pltpu.VMEM((C, D), jnp.float32) for _ in range(2 * R)]
            + [pltpu.VMEM_SHARED((N, D), jnp.float32)]
            + [pltpu.SemaphoreType.DMA for _ in range(5 * R)]
        ),
    )


def _dense_body(x_ref, agg_ref, eps_ref, w1_ref, b1_ref, g_ref, be_ref,
                w2_ref, b2_ref, o_ref):
    h = (1.0 + eps_ref[0, 0]) * x_ref[...] + agg_ref[0:N] + agg_ref[N:2 * N]
    h1 = jnp.dot(h, w1_ref[...], preferred_element_type=jnp.float32) + b1_ref[...]
    mu = jnp.mean(h1, axis=0, keepdims=True)
    var = jnp.mean(jnp.square(h1 - mu), axis=0, keepdims=True)
    hn = (h1 - mu) * (g_ref[...] * lax.rsqrt(var + 1e-5)) + be_ref[...]
    hr = jnp.maximum(hn, 0.0)
    h2 = jnp.dot(hr, w2_ref[...], preferred_element_type=jnp.float32) + b2_ref[...]
    o_ref[...] = jnp.maximum(h2, 0.0)


_dense = pl.pallas_call(
    _dense_body,
    out_shape=jax.ShapeDtypeStruct((N, D), jnp.float32),
)


def _dense_final_body(x_ref, agg_ref, eps_ref, w1_ref, b1_ref, g_ref, be_ref,
                      w2_ref, b2_ref, h0_ref, hp1_ref, wl_ref, bl_ref, o_ref):
    h = (1.0 + eps_ref[0, 0]) * x_ref[...] + agg_ref[0:N] + agg_ref[N:2 * N]
    h1 = jnp.dot(h, w1_ref[...], preferred_element_type=jnp.float32) + b1_ref[...]
    mu = jnp.mean(h1, axis=0, keepdims=True)
    var = jnp.mean(jnp.square(h1 - mu), axis=0, keepdims=True)
    hn = (h1 - mu) * (g_ref[...] * lax.rsqrt(var + 1e-5)) + be_ref[...]
    hr = jnp.maximum(hn, 0.0)
    h2 = jnp.dot(hr, w2_ref[...], preferred_element_type=jnp.float32) + b2_ref[...]
    h3 = jnp.maximum(h2, 0.0)
    acc = jnp.dot(h0_ref[...], wl_ref[0:D], preferred_element_type=jnp.float32)
    acc += jnp.dot(hp1_ref[...], wl_ref[D:2 * D], preferred_element_type=jnp.float32)
    acc += jnp.dot(x_ref[...], wl_ref[2 * D:3 * D], preferred_element_type=jnp.float32)
    acc += jnp.dot(h3, wl_ref[3 * D:4 * D], preferred_element_type=jnp.float32)
    o_ref[...] = acc + bl_ref[...]


_dense_final = pl.pallas_call(
    _dense_final_body,
    out_shape=jax.ShapeDtypeStruct((N, D), jnp.float32),
)


def kernel(x, edge_index, edge_attr,
           W1_0, b1_0, gamma_0, beta_0, W2_0, b2_0, eps_0,
           W1_1, b1_1, gamma_1, beta_1, W2_1, b2_1, eps_1,
           W1_2, b1_2, gamma_2, beta_2, W2_2, b2_2, eps_2,
           W_lin, b_lin):
    src = edge_index[0]
    dst = edge_index[1]
    params = [
        (W1_0, b1_0, gamma_0, beta_0, W2_0, b2_0, eps_0),
        (W1_1, b1_1, gamma_1, beta_1, W2_1, b2_1, eps_1),
        (W1_2, b1_2, gamma_2, beta_2, W2_2, b2_2, eps_2),
    ]
    h = x
    hs = [x]
    for l in range(2):
        W1, b1, gamma, beta, W2, b2, eps = params[l]
        agg = _get_edge_agg()(h, src, dst, edge_attr)
        h = _dense(h, agg, eps.reshape(1, 1), W1, b1.reshape(1, 2 * D),
                   gamma.reshape(1, 2 * D), beta.reshape(1, 2 * D),
                   W2, b2.reshape(1, D))
        hs.append(h)
    W1, b1, gamma, beta, W2, b2, eps = params[2]
    agg = _get_edge_agg()(h, src, dst, edge_attr)
    return _dense_final(h, agg, eps.reshape(1, 1), W1, b1.reshape(1, 2 * D),
                        gamma.reshape(1, 2 * D), beta.reshape(1, 2 * D),
                        W2, b2.reshape(1, D), hs[0], hs[1],
                        W_lin, b_lin.reshape(1, D))


# speedup vs baseline: 3.7886x; 1.0193x over previous
import functools

import jax
import jax.numpy as jnp
from jax import lax
from jax.experimental import pallas as pl
from jax.experimental.pallas import tpu as pltpu
from jax.experimental.pallas import tpu_sc as plsc

N = 10000
E = 320000
D = 128
NC = 2
NS = 16
NW = NC * NS
EPT = E // NW
C = 40
NCHUNK = EPT // C
R = 4
ZCH = 40
NZ = N // ZCH
VPR = D // 16


def _edge_body(h_hbm, src_hbm, dst_hbm, attr_hbm, out_hbm, *rest):
    sib = rest[0:R]
    dib = rest[R:2 * R]
    gbufs = rest[2 * R:3 * R]
    abufs = rest[3 * R:4 * R]
    agg_sh = rest[4 * R]
    zb = rest[4 * R + 1]
    zsem = rest[4 * R + 2]
    isems = rest[4 * R + 3:5 * R + 3]
    dsems = rest[5 * R + 3:6 * R + 3]
    gsems = rest[6 * R + 3:7 * R + 3]
    asems = rest[7 * R + 3:8 * R + 3]
    ssems = rest[8 * R + 3:9 * R + 3]

    c = lax.axis_index("c")
    s = lax.axis_index("s")
    wid = s * NC + c
    base = wid * EPT
    nz_mine = (NZ - s + NS - 1) // NS

    def _i_sidx(q, r):
        pltpu.async_copy(src_hbm.at[pl.ds(base + q * C, C)], sib[r], isems[r])

    def _w_sidx(q, r):
        pltpu.make_async_copy(src_hbm.at[pl.ds(base + q * C, C)],
                              sib[r], isems[r]).wait()

    def _i_didx(q, r):
        pltpu.async_copy(dst_hbm.at[pl.ds(base + q * C, C)], dib[r], dsems[r])

    def _w_didx(q, r):
        pltpu.make_async_copy(dst_hbm.at[pl.ds(base + q * C, C)],
                              dib[r], dsems[r]).wait()

    def _i_attr(q, r):
        pltpu.async_copy(attr_hbm.at[pl.ds(base + q * C, C)], abufs[r], asems[r])

    def _w_attr(q, r):
        pltpu.make_async_copy(attr_hbm.at[pl.ds(base + q * C, C)],
                              abufs[r], asems[r]).wait()

    def _i_gather(r):
        pltpu.async_copy(h_hbm.at[sib[r]], gbufs[r], gsems[r])

    def _w_gather(r):
        pltpu.make_async_copy(h_hbm.at[sib[r]], gbufs[r], gsems[r]).wait()

    def _i_scatter(r):
        pltpu.async_copy(gbufs[r], agg_sh.at[dib[r]], ssems[r], add=True)

    def _w_scatter(r):
        pltpu.make_async_copy(gbufs[r], agg_sh.at[dib[r]], ssems[r]).wait()

    def _step(q, r, pre3, pre2, w_sc):
        r3 = (r + 3) % R
        r2 = (r + 2) % R
        if pre3:
            _i_sidx(q + 3, r3)
            _i_attr(q + 3, r3)
        if w_sc:
            _w_scatter(r2)
        if pre2:
            _i_didx(q + 2, r2)
            _w_sidx(q + 2, r2)
            _i_gather(r2)
        _w_gather(r)
        _w_attr(q, r)
        gbuf, abuf = gbufs[r], abufs[r]

        def _row(i, rc):
            for j in range(VPR):
                v = gbuf[i, pl.ds(j * 16, 16)] + abuf[i, pl.ds(j * 16, 16)]
                gbuf[i, pl.ds(j * 16, 16)] = jnp.maximum(v, 0.0)
            return rc
        lax.fori_loop(0, C, _row, 0)
        _w_didx(q, r)
        _i_scatter(r)

    _i_sidx(0, 0)
    _i_sidx(1, 1)
    _i_sidx(2, 2)
    _i_didx(0, 0)
    _i_didx(1, 1)
    _i_attr(0, 0)
    _i_attr(1, 1)
    _i_attr(2, 2)
    _w_sidx(0, 0)
    _i_gather(0)
    _w_sidx(1, 1)
    _i_gather(1)

    def _zrow(i, carry):
        for j in range(VPR):
            zb[i, pl.ds(j * 16, 16)] = jnp.zeros((16,), jnp.float32)
        return carry
    lax.fori_loop(0, ZCH, _zrow, 0)

    def _zcopy(k, carry):
        r = (s + k * NS) * ZCH
        pltpu.async_copy(zb, agg_sh.at[pl.ds(r, ZCH)], zsem)
        return carry
    lax.fori_loop(0, nz_mine, _zcopy, 0)

    def _zdrain(k, carry):
        r = (s + k * NS) * ZCH
        pltpu.make_async_copy(zb, agg_sh.at[pl.ds(r, ZCH)], zsem).wait()
        return carry
    lax.fori_loop(0, nz_mine, _zdrain, 0)
    plsc.subcore_barrier()

    _step(0, 0, True, True, False)
    _step(1, 1, True, True, False)
    _step(2, 2, True, True, True)
    _step(3, 3, True, True, True)

    def _super(j, carry):
        q0 = R * j
        for i in range(R):
            _step(q0 + i, i, True, True, True)
        return carry
    lax.fori_loop(1, NCHUNK // R - 1, _super, 0)
    for q in range((NCHUNK // R - 1) * R, NCHUNK):
        _step(q, q % R, q + 3 < NCHUNK, q + 2 < NCHUNK, True)
    _w_scatter((NCHUNK - 2) % R)
    _w_scatter((NCHUNK - 1) % R)
    plsc.subcore_barrier()

    osem = isems[1]

    def _ocopy(k, carry):
        r = (s + k * NS) * ZCH
        pltpu.async_copy(agg_sh.at[pl.ds(r, ZCH)],
                         out_hbm.at[pl.ds(c * N + r, ZCH)], osem)
        return carry
    lax.fori_loop(0, nz_mine, _ocopy, 0)

    def _odrain(k, carry):
        r = (s + k * NS) * ZCH
        pltpu.make_async_copy(agg_sh.at[pl.ds(r, ZCH)],
                              out_hbm.at[pl.ds(c * N + r, ZCH)], osem).wait()
        return carry
    lax.fori_loop(0, nz_mine, _odrain, 0)


@functools.lru_cache(maxsize=None)
def _get_edge_agg():
    return pl.kernel(
        _edge_body,
        out_type=jax.ShapeDtypeStruct((2 * N, D), jnp.float32),
        mesh=plsc.VectorSubcoreMesh(core_axis_name="c", subcore_axis_name="s"),
        scratch_types=(
            [pltpu.VMEM((C,), jnp.int32) for _ in range(2 * R)]
            + [pltpu.VMEM((C, D), jnp.float32) for _ in range(2 * R)]
            + [pltpu.VMEM_SHARED((N, D), jnp.float32)]
            + [pltpu.VMEM((ZCH, D), jnp.float32)]
            + [pltpu.SemaphoreType.DMA]
            + [pltpu.SemaphoreType.DMA for _ in range(5 * R)]
        ),
    )


def _dense_body(x_ref, agg_ref, eps_ref, w1_ref, b1_ref, g_ref, be_ref,
                w2_ref, b2_ref, o_ref):
    h = (1.0 + eps_ref[0, 0]) * x_ref[...] + agg_ref[0:N] + agg_ref[N:2 * N]
    h1 = jnp.dot(h, w1_ref[...], preferred_element_type=jnp.float32) + b1_ref[...]
    mu = jnp.mean(h1, axis=0, keepdims=True)
    var = jnp.mean(jnp.square(h1 - mu), axis=0, keepdims=True)
    hn = (h1 - mu) * (g_ref[...] * lax.rsqrt(var + 1e-5)) + be_ref[...]
    hr = jnp.maximum(hn, 0.0)
    h2 = jnp.dot(hr, w2_ref[...], preferred_element_type=jnp.float32) + b2_ref[...]
    o_ref[...] = jnp.maximum(h2, 0.0)


_dense = pl.pallas_call(
    _dense_body,
    out_shape=jax.ShapeDtypeStruct((N, D), jnp.float32),
)


def _dense_final_body(x_ref, agg_ref, eps_ref, w1_ref, b1_ref, g_ref, be_ref,
                      w2_ref, b2_ref, h0_ref, hp1_ref, wl_ref, bl_ref, o_ref):
    h = (1.0 + eps_ref[0, 0]) * x_ref[...] + agg_ref[0:N] + agg_ref[N:2 * N]
    h1 = jnp.dot(h, w1_ref[...], preferred_element_type=jnp.float32) + b1_ref[...]
    mu = jnp.mean(h1, axis=0, keepdims=True)
    var = jnp.mean(jnp.square(h1 - mu), axis=0, keepdims=True)
    hn = (h1 - mu) * (g_ref[...] * lax.rsqrt(var + 1e-5)) + be_ref[...]
    hr = jnp.maximum(hn, 0.0)
    h2 = jnp.dot(hr, w2_ref[...], preferred_element_type=jnp.float32) + b2_ref[...]
    h3 = jnp.maximum(h2, 0.0)
    acc = jnp.dot(h0_ref[...], wl_ref[0:D], preferred_element_type=jnp.float32)
    acc += jnp.dot(hp1_ref[...], wl_ref[D:2 * D], preferred_element_type=jnp.float32)
    acc += jnp.dot(x_ref[...], wl_ref[2 * D:3 * D], preferred_element_type=jnp.float32)
    acc += jnp.dot(h3, wl_ref[3 * D:4 * D], preferred_element_type=jnp.float32)
    o_ref[...] = acc + bl_ref[...]


_dense_final = pl.pallas_call(
    _dense_final_body,
    out_shape=jax.ShapeDtypeStruct((N, D), jnp.float32),
)


def kernel(x, edge_index, edge_attr,
           W1_0, b1_0, gamma_0, beta_0, W2_0, b2_0, eps_0,
           W1_1, b1_1, gamma_1, beta_1, W2_1, b2_1, eps_1,
           W1_2, b1_2, gamma_2, beta_2, W2_2, b2_2, eps_2,
           W_lin, b_lin):
    src = edge_index[0]
    dst = edge_index[1]
    params = [
        (W1_0, b1_0, gamma_0, beta_0, W2_0, b2_0, eps_0),
        (W1_1, b1_1, gamma_1, beta_1, W2_1, b2_1, eps_1),
        (W1_2, b1_2, gamma_2, beta_2, W2_2, b2_2, eps_2),
    ]
    h = x
    hs = [x]
    for l in range(2):
        W1, b1, gamma, beta, W2, b2, eps = params[l]
        agg = _get_edge_agg()(h, src, dst, edge_attr)
        h = _dense(h, agg, eps.reshape(1, 1), W1, b1.reshape(1, 2 * D),
                   gamma.reshape(1, 2 * D), beta.reshape(1, 2 * D),
                   W2, b2.reshape(1, D))
        hs.append(h)
    W1, b1, gamma, beta, W2, b2, eps = params[2]
    agg = _get_edge_agg()(h, src, dst, edge_attr)
    return _dense_final(h, agg, eps.reshape(1, 1), W1, b1.reshape(1, 2 * D),
                        gamma.reshape(1, 2 * D), beta.reshape(1, 2 * D),
                        W2, b2.reshape(1, D), hs[0], hs[1],
                        W_lin, b_lin.reshape(1, D))


# speedup vs baseline: 3.8088x; 1.0053x over previous
import functools

import jax
import jax.numpy as jnp
from jax import lax
from jax.experimental import pallas as pl
from jax.experimental.pallas import tpu as pltpu
from jax.experimental.pallas import tpu_sc as plsc

N = 10000
E = 320000
D = 128
NC = 2
NS = 16
NW = NC * NS
EPT = E // NW
C = 40
NCHUNK = EPT // C
RG = 6
RA = 3
ZCH = 40
NZ = N // ZCH
VPR = D // 16


def _edge_body(h_hbm, src_hbm, dst_hbm, attr_hbm, out_hbm, *rest):
    sib = rest[0:RG]
    dib = rest[RG:2 * RG]
    gbufs = rest[2 * RG:3 * RG]
    abufs = rest[3 * RG:3 * RG + RA]
    agg_sh = rest[3 * RG + RA]
    zsem = rest[3 * RG + RA + 1]
    o = 3 * RG + RA + 2
    isems = rest[o:o + RG]
    dsems = rest[o + RG:o + 2 * RG]
    gsems = rest[o + 2 * RG:o + 3 * RG]
    asems = rest[o + 3 * RG:o + 3 * RG + RA]
    ssems = rest[o + 3 * RG + RA:o + 4 * RG + RA]

    c = lax.axis_index("c")
    s = lax.axis_index("s")
    wid = s * NC + c
    base = wid * EPT
    nz_mine = (NZ - s + NS - 1) // NS

    def _i_sidx(q, r):
        pltpu.async_copy(src_hbm.at[pl.ds(base + q * C, C)], sib[r], isems[r])

    def _w_sidx(q, r):
        pltpu.make_async_copy(src_hbm.at[pl.ds(base + q * C, C)],
                              sib[r], isems[r]).wait()

    def _i_didx(q, r):
        pltpu.async_copy(dst_hbm.at[pl.ds(base + q * C, C)], dib[r], dsems[r])

    def _w_didx(q, r):
        pltpu.make_async_copy(dst_hbm.at[pl.ds(base + q * C, C)],
                              dib[r], dsems[r]).wait()

    def _i_attr(q, r):
        pltpu.async_copy(attr_hbm.at[pl.ds(base + q * C, C)], abufs[r], asems[r])

    def _w_attr(q, r):
        pltpu.make_async_copy(attr_hbm.at[pl.ds(base + q * C, C)],
                              abufs[r], asems[r]).wait()

    def _i_gather(r):
        pltpu.async_copy(h_hbm.at[sib[r]], gbufs[r], gsems[r])

    def _w_gather(r):
        pltpu.make_async_copy(h_hbm.at[sib[r]], gbufs[r], gsems[r]).wait()

    def _i_scatter(r):
        pltpu.async_copy(gbufs[r], agg_sh.at[dib[r]], ssems[r], add=True)

    def _w_scatter(r):
        pltpu.make_async_copy(gbufs[r], agg_sh.at[dib[r]], ssems[r]).wait()

    def _step(q, r, ra, pre4, pre3, w_sc):
        r4 = (r + 4) % RG
        r3 = (r + 3) % RG
        if pre4:
            _i_sidx(q + 4, r4)
        if w_sc:
            _w_scatter(r3)
        if pre3:
            _i_didx(q + 3, r3)
            _w_sidx(q + 3, r3)
            _i_gather(r3)
        _w_gather(r)
        _w_attr(q, ra)
        gbuf, abuf = gbufs[r], abufs[ra]

        def _row(i, rc):
            for j in range(VPR):
                v = gbuf[i, pl.ds(j * 16, 16)] + abuf[i, pl.ds(j * 16, 16)]
                gbuf[i, pl.ds(j * 16, 16)] = jnp.maximum(v, 0.0)
            return rc
        lax.fori_loop(0, C, _row, 0)
        if pre3:
            _i_attr(q + 3, ra)
        _w_didx(q, r)
        _i_scatter(r)

    _i_sidx(0, 0)
    _i_sidx(1, 1)
    _i_sidx(2, 2)
    _i_sidx(3, 3)
    _i_didx(0, 0)
    _i_didx(1, 1)
    _i_didx(2, 2)
    _w_sidx(0, 0)
    _i_gather(0)
    _w_sidx(1, 1)
    _i_gather(1)
    _w_sidx(2, 2)
    _i_gather(2)

    zb = abufs[0]

    def _zrow(i, carry):
        for j in range(VPR):
            zb[i, pl.ds(j * 16, 16)] = jnp.zeros((16,), jnp.float32)
        return carry
    lax.fori_loop(0, ZCH, _zrow, 0)

    def _zcopy(k, carry):
        r = (s + k * NS) * ZCH
        pltpu.async_copy(zb, agg_sh.at[pl.ds(r, ZCH)], zsem)
        return carry
    lax.fori_loop(0, nz_mine, _zcopy, 0)

    def _zdrain(k, carry):
        r = (s + k * NS) * ZCH
        pltpu.make_async_copy(zb, agg_sh.at[pl.ds(r, ZCH)], zsem).wait()
        return carry
    lax.fori_loop(0, nz_mine, _zdrain, 0)
    _i_attr(0, 0)
    _i_attr(1, 1)
    _i_attr(2, 2)
    plsc.subcore_barrier()

    _step(0, 0, 0, True, True, False)
    _step(1, 1, 1, True, True, False)
    _step(2, 2, 2, True, True, False)
    _step(3, 3, 0, True, True, True)
    _step(4, 4, 1, True, True, True)
    _step(5, 5, 2, True, True, True)

    def _super(j, carry):
        q0 = RG * j
        for i in range(RG):
            _step(q0 + i, i, i % RA, True, True, True)
        return carry
    lax.fori_loop(1, NCHUNK // RG, _super, 0)
    for q in range((NCHUNK // RG) * RG, NCHUNK):
        _step(q, q % RG, q % RA, q + 4 < NCHUNK, q + 3 < NCHUNK, True)
    _w_scatter((NCHUNK - 3) % RG)
    _w_scatter((NCHUNK - 2) % RG)
    _w_scatter((NCHUNK - 1) % RG)
    plsc.subcore_barrier()

    osem = isems[1]

    def _ocopy(k, carry):
        r = (s + k * NS) * ZCH
        pltpu.async_copy(agg_sh.at[pl.ds(r, ZCH)],
                         out_hbm.at[pl.ds(c * N + r, ZCH)], osem)
        return carry
    lax.fori_loop(0, nz_mine, _ocopy, 0)

    def _odrain(k, carry):
        r = (s + k * NS) * ZCH
        pltpu.make_async_copy(agg_sh.at[pl.ds(r, ZCH)],
                              out_hbm.at[pl.ds(c * N + r, ZCH)], osem).wait()
        return carry
    lax.fori_loop(0, nz_mine, _odrain, 0)


@functools.lru_cache(maxsize=None)
def _get_edge_agg():
    return pl.kernel(
        _edge_body,
        out_type=jax.ShapeDtypeStruct((2 * N, D), jnp.float32),
        mesh=plsc.VectorSubcoreMesh(core_axis_name="c", subcore_axis_name="s"),
        scratch_types=(
            [pltpu.VMEM((C,), jnp.int32) for _ in range(2 * RG)]
            + [pltpu.VMEM((C, D), jnp.float32) for _ in range(3 * RG + RA - 2 * RG)]
            + [pltpu.VMEM_SHARED((N, D), jnp.float32)]
            + [pltpu.SemaphoreType.DMA]
            + [pltpu.SemaphoreType.DMA for _ in range(4 * RG + RA)]
        ),
    )


def _dense_body(x_ref, agg_ref, eps_ref, w1_ref, b1_ref, g_ref, be_ref,
                w2_ref, b2_ref, o_ref):
    h = (1.0 + eps_ref[0, 0]) * x_ref[...] + agg_ref[0:N] + agg_ref[N:2 * N]
    h1 = jnp.dot(h, w1_ref[...], preferred_element_type=jnp.float32) + b1_ref[...]
    mu = jnp.mean(h1, axis=0, keepdims=True)
    var = jnp.mean(jnp.square(h1 - mu), axis=0, keepdims=True)
    hn = (h1 - mu) * (g_ref[...] * lax.rsqrt(var + 1e-5)) + be_ref[...]
    hr = jnp.maximum(hn, 0.0)
    h2 = jnp.dot(hr, w2_ref[...], preferred_element_type=jnp.float32) + b2_ref[...]
    o_ref[...] = jnp.maximum(h2, 0.0)


_dense = pl.pallas_call(
    _dense_body,
    out_shape=jax.ShapeDtypeStruct((N, D), jnp.float32),
)


def _dense_final_body(x_ref, agg_ref, eps_ref, w1_ref, b1_ref, g_ref, be_ref,
                      w2_ref, b2_ref, h0_ref, hp1_ref, wl_ref, bl_ref, o_ref):
    h = (1.0 + eps_ref[0, 0]) * x_ref[...] + agg_ref[0:N] + agg_ref[N:2 * N]
    h1 = jnp.dot(h, w1_ref[...], preferred_element_type=jnp.float32) + b1_ref[...]
    mu = jnp.mean(h1, axis=0, keepdims=True)
    var = jnp.mean(jnp.square(h1 - mu), axis=0, keepdims=True)
    hn = (h1 - mu) * (g_ref[...] * lax.rsqrt(var + 1e-5)) + be_ref[...]
    hr = jnp.maximum(hn, 0.0)
    h2 = jnp.dot(hr, w2_ref[...], preferred_element_type=jnp.float32) + b2_ref[...]
    h3 = jnp.maximum(h2, 0.0)
    acc = jnp.dot(h0_ref[...], wl_ref[0:D], preferred_element_type=jnp.float32)
    acc += jnp.dot(hp1_ref[...], wl_ref[D:2 * D], preferred_element_type=jnp.float32)
    acc += jnp.dot(x_ref[...], wl_ref[2 * D:3 * D], preferred_element_type=jnp.float32)
    acc += jnp.dot(h3, wl_ref[3 * D:4 * D], preferred_element_type=jnp.float32)
    o_ref[...] = acc + bl_ref[...]


_dense_final = pl.pallas_call(
    _dense_final_body,
    out_shape=jax.ShapeDtypeStruct((N, D), jnp.float32),
)


def kernel(x, edge_index, edge_attr,
           W1_0, b1_0, gamma_0, beta_0, W2_0, b2_0, eps_0,
           W1_1, b1_1, gamma_1, beta_1, W2_1, b2_1, eps_1,
           W1_2, b1_2, gamma_2, beta_2, W2_2, b2_2, eps_2,
           W_lin, b_lin):
    src = edge_index[0]
    dst = edge_index[1]
    params = [
        (W1_0, b1_0, gamma_0, beta_0, W2_0, b2_0, eps_0),
        (W1_1, b1_1, gamma_1, beta_1, W2_1, b2_1, eps_1),
        (W1_2, b1_2, gamma_2, beta_2, W2_2, b2_2, eps_2),
    ]
    h = x
    hs = [x]
    for l in range(2):
        W1, b1, gamma, beta, W2, b2, eps = params[l]
        agg = _get_edge_agg()(h, src, dst, edge_attr)
        h = _dense(h, agg, eps.reshape(1, 1), W1, b1.reshape(1, 2 * D),
                   gamma.reshape(1, 2 * D), beta.reshape(1, 2 * D),
                   W2, b2.reshape(1, D))
        hs.append(h)
    W1, b1, gamma, beta, W2, b2, eps = params[2]
    agg = _get_edge_agg()(h, src, dst, edge_attr)
    return _dense_final(h, agg, eps.reshape(1, 1), W1, b1.reshape(1, 2 * D),
                        gamma.reshape(1, 2 * D), beta.reshape(1, 2 * D),
                        W2, b2.reshape(1, D), hs[0], hs[1],
                        W_lin, b_lin.reshape(1, D))
